# baseline (speedup 1.0000x reference)
import jax
import jax.numpy as jnp
from jax import lax
from jax.experimental import pallas as pl
from jax.experimental.pallas import tpu as pltpu
from jax.experimental.pallas import tpu_sc as plsc

_NC, _NS = 2, 16
_NW = _NC * _NS
_D = 128
_L = 16
_ROW0, _NROW = 17, 51
_CB = 8
_NAU = 8

_GROUPS = (
    (0, 10, 1.0 / 10.0, (0, 1, 2)),
    (19, 31, 1.0 / 12.0, (3,)),
    (10, 19, 1.0 / 9.0, (4,)),
    (31, 51, 1.0 / 20.0, (5, 6, 7)),
)


def _tree_sum(vals):
    while len(vals) > 1:
        nxt = [vals[i] + vals[i + 1] for i in range(0, len(vals) - 1, 2)]
        if len(vals) % 2:
            nxt.append(vals[-1])
        vals = nxt
    return vals[0]


def _body(x_hbm, out_hbm, xv, ov, in_sems, out_sems):
    batches = x_hbm.shape[1]
    per_w = batches // _NW
    nchunk = per_w // _CB
    wid = lax.axis_index("s") * _NC + lax.axis_index("c")
    base = wid * per_w

    def in_copy(cc, par):
        b0 = base + cc * _CB
        return pltpu.make_async_copy(
            x_hbm.at[pl.ds(_ROW0, _NROW), pl.ds(b0, _CB)],
            xv.at[par],
            in_sems.at[par],
        )

    def out_copy(cc, par):
        b0 = base + cc * _CB
        return pltpu.make_async_copy(
            ov.at[par],
            out_hbm.at[pl.ds(b0, _CB)],
            out_sems.at[par],
        )

    def compute(par):
        for b in range(_CB):
            def k_step(k, carry):
                sl = pl.ds(k * _L, _L)
                for lo, hi, scale, outs in _GROUPS:
                    acc = _tree_sum([xv[par, r, b, sl] for r in range(lo, hi)])
                    acc = acc * scale
                    for o in outs:
                        ov[par, b, o, sl] = acc
                return carry

            lax.fori_loop(0, _D // _L, k_step, 0, unroll=2)

    in_copy(0, 0).start()
    in_copy(1, 1).start()

    def chunk_pair(i, carry):
        c = i * 2
        for par in range(2):
            cc = c + par
            in_copy(cc, par).wait()

            @pl.when(cc >= 2)
            def _():
                out_copy(cc - 2, par).wait()

            compute(par)
            out_copy(cc, par).start()

            @pl.when(cc + 2 < nchunk)
            def _():
                in_copy(cc + 2, par).start()

        return carry

    lax.fori_loop(0, nchunk // 2, chunk_pair, 0)
    out_copy(nchunk - 2, 0).wait()
    out_copy(nchunk - 1, 1).wait()


def kernel(node_feats):
    batches, rows, d = node_feats.shape
    assert rows == 68 and d == _D
    assert batches % (_NW * _CB * 2) == 0

    x_t = jnp.transpose(node_feats, (1, 0, 2))

    mesh = plsc.VectorSubcoreMesh(
        core_axis_name="c", subcore_axis_name="s", num_cores=_NC, num_subcores=_NS
    )
    f = pl.kernel(
        _body,
        out_type=jax.ShapeDtypeStruct((batches, _NAU, _D), jnp.float32),
        mesh=mesh,
        scratch_types=[
            pltpu.VMEM((2, _NROW, _CB, _D), jnp.float32),
            pltpu.VMEM((2, _CB, _NAU, _D), jnp.float32),
            pltpu.SemaphoreType.DMA((2,)),
            pltpu.SemaphoreType.DMA((2,)),
        ],
    )
    return f(x_t)


# speedup vs baseline: 1.2902x; 1.2902x over previous
import jax
import jax.numpy as jnp
from jax import lax
from jax.experimental import pallas as pl
from jax.experimental.pallas import tpu as pltpu
from jax.experimental.pallas import tpu_sc as plsc

_NC, _NS = 2, 16
_NW = _NC * _NS
_D = 128
_L = 16
_ROW0, _NROW = 17, 51
_CB = 8
_NAU = 8

_GROUPS = (
    (0, 10, 1.0 / 10.0, (0, 1, 2)),
    (19, 31, 1.0 / 12.0, (3,)),
    (10, 19, 1.0 / 9.0, (4,)),
    (31, 51, 1.0 / 20.0, (5, 6, 7)),
)


def _tree_sum(vals):
    while len(vals) > 1:
        nxt = [vals[i] + vals[i + 1] for i in range(0, len(vals) - 1, 2)]
        if len(vals) % 2:
            nxt.append(vals[-1])
        vals = nxt
    return vals[0]


def _body(x_hbm, out_hbm, xv, ov, in_sems, out_sems):
    batches = x_hbm.shape[1]
    per_w = batches // _NW
    nchunk = per_w // _CB
    wid = lax.axis_index("s") * _NC + lax.axis_index("c")
    base = wid * per_w

    def in_copy(cc, par):
        b0 = base + cc * _CB
        return pltpu.make_async_copy(
            x_hbm.at[pl.ds(_ROW0, _NROW), pl.ds(b0, _CB)],
            xv.at[par],
            in_sems.at[par],
        )

    def out_copy(cc, par):
        b0 = base + cc * _CB
        return pltpu.make_async_copy(
            ov.at[par],
            out_hbm.at[pl.ds(b0, _CB)],
            out_sems.at[par],
        )

    def compute(par):
        for b in range(_CB):
            def k_step(k, carry):
                sl = pl.ds(k * _L, _L)
                for lo, hi, scale, outs in _GROUPS:
                    acc = _tree_sum([xv[par, r, b, sl] for r in range(lo, hi)])
                    acc = acc * scale
                    for o in outs:
                        ov[par, b, o, sl] = acc
                return carry

            lax.fori_loop(0, _D // _L, k_step, 0)

    in_copy(0, 0).start()
    in_copy(1, 1).start()

    def chunk_pair(i, carry):
        c = i * 2
        for par in range(2):
            cc = c + par
            in_copy(cc, par).wait()

            @pl.when(cc >= 2)
            def _():
                out_copy(cc - 2, par).wait()

            compute(par)
            out_copy(cc, par).start()

            @pl.when(cc + 2 < nchunk)
            def _():
                in_copy(cc + 2, par).start()

        return carry

    lax.fori_loop(0, nchunk // 2, chunk_pair, 0)
    out_copy(nchunk - 2, 0).wait()
    out_copy(nchunk - 1, 1).wait()


def kernel(node_feats):
    batches, rows, d = node_feats.shape
    assert rows == 68 and d == _D
    assert batches % (_NW * _CB * 2) == 0

    x_t = jnp.transpose(node_feats, (1, 0, 2))

    mesh = plsc.VectorSubcoreMesh(
        core_axis_name="c", subcore_axis_name="s", num_cores=_NC, num_subcores=_NS
    )
    f = pl.kernel(
        _body,
        out_type=jax.ShapeDtypeStruct((batches, _NAU, _D), jnp.float32),
        mesh=mesh,
        scratch_types=[
            pltpu.VMEM((2, _NROW, _CB, _D), jnp.float32),
            pltpu.VMEM((2, _CB, _NAU, _D), jnp.float32),
            pltpu.SemaphoreType.DMA((2,)),
            pltpu.SemaphoreType.DMA((2,)),
        ],
    )
    return f(x_t)
